# baseline (speedup 1.0000x reference)
import functools

import jax
import jax.numpy as jnp
from jax import lax
from jax.experimental import pallas as pl
from jax.experimental.pallas import tpu as pltpu
from jax.experimental.pallas import tpu_sc as plsc

_INPUT_DIM = 768
_NUM_EXPERTS = 64
_BLOCK_T = 4096
_CHUNK = 256
_GRP = 16

_NW = 32
_TOKENS = 32768
_HALF = _TOKENS // 2
_TPW = _HALF // _NW
_GPW = _TPW // _GRP


def _dot(a, bm):
    return jax.lax.dot_general(
        a, bm,
        dimension_numbers=(((1,), (0,)), ((), ())),
        preferred_element_type=jnp.float32,
    )


def _router_block(x0_ref, x1_ref, x2_ref, wt_ref, b_ref, pk_ref):
    w = wt_ref[...]
    logits = (_dot(x0_ref[...], w[0:_CHUNK, :])
              + _dot(x1_ref[...], w[_CHUNK:2 * _CHUNK, :])
              + _dot(x2_ref[...], w[2 * _CHUNK:3 * _CHUNK, :])
              + b_ref[...])
    v1 = jnp.max(logits, axis=1, keepdims=True)
    m1 = logits == v1
    masked = jnp.where(m1, -jnp.inf, logits)
    v2 = jnp.max(masked, axis=1, keepdims=True)
    m2 = masked == v2
    t = jnp.exp(v2 - v1)
    w1 = 1.0 / (1.0 + t)
    w2 = t * w1
    colf = jax.lax.broadcasted_iota(
        jnp.int32, (_NUM_EXPERTS, 1), 0).astype(jnp.float32)
    i1f = _dot(m1.astype(jnp.float32), colf)
    i2f = _dot(m2.astype(jnp.float32), colf)
    g = _BLOCK_T // _GRP
    pk_ref[...] = jnp.concatenate(
        [w1.reshape(g, _GRP), w2.reshape(g, _GRP),
         i1f.reshape(g, _GRP), i2f.reshape(g, _GRP),
         jnp.zeros((g, 64), jnp.float32)], axis=1)


def _router(x, wt, b2, h):
    grid = (_HALF // _BLOCK_T,)
    nb = _HALF // _BLOCK_T
    xspec = lambda j: pl.BlockSpec(
        (_BLOCK_T, _CHUNK), lambda i, j=j, h=h: (i + h * nb, j))
    return pl.pallas_call(
        _router_block,
        grid=grid,
        in_specs=[
            xspec(0),
            xspec(1),
            xspec(2),
            pl.BlockSpec((_INPUT_DIM, _NUM_EXPERTS), lambda i: (0, 0)),
            pl.BlockSpec((1, _NUM_EXPERTS), lambda i: (0, 0)),
        ],
        out_specs=pl.BlockSpec((_BLOCK_T // _GRP, 128), lambda i: (i, 0)),
        out_shape=jax.ShapeDtypeStruct((_HALF // _GRP, 128), jnp.float32),
        compiler_params=pltpu.CompilerParams(
            dimension_semantics=("parallel",),
        ),
    )(x, x, x, wt, b2)


@functools.partial(
    pl.kernel,
    out_type=jax.ShapeDtypeStruct((_HALF, _NUM_EXPERTS), jnp.float32),
    mesh=plsc.VectorSubcoreMesh(core_axis_name="c", subcore_axis_name="s"),
    scratch_types=[
        pltpu.VMEM((_GPW, 128), jnp.float32),
        pltpu.VMEM((_TPW, _NUM_EXPERTS), jnp.float32),
    ],
    compiler_params=pltpu.CompilerParams(needs_layout_passes=False),
)
def _sc_expand(pk_hbm, out_hbm, pk_v, rows_v):
    wid = lax.axis_index("s") * 2 + lax.axis_index("c")
    pltpu.sync_copy(pk_hbm.at[pl.ds(wid * _GPW, _GPW), :], pk_v)
    zvec = jnp.zeros((_GRP,), jnp.float32)
    iota = lax.iota(jnp.int32, 16)

    def zero_body(t, carry):
        rows_v[t, pl.ds(0, 16)] = zvec
        rows_v[t, pl.ds(16, 16)] = zvec
        rows_v[t, pl.ds(32, 16)] = zvec
        rows_v[t, pl.ds(48, 16)] = zvec
        return carry

    lax.fori_loop(0, _TPW, zero_body, 0)

    def scat_body(g, carry):
        w1 = pk_v[g, pl.ds(0, 16)]
        w2 = pk_v[g, pl.ds(16, 16)]
        i1 = pk_v[g, pl.ds(32, 16)].astype(jnp.int32)
        i2 = pk_v[g, pl.ds(48, 16)].astype(jnp.int32)
        row = g * _GRP + iota
        plsc.store_scatter(rows_v, [row, i1], w1)
        plsc.store_scatter(rows_v, [row, i2], w2)
        return carry

    lax.fori_loop(0, _GPW, scat_body, 0)
    pltpu.sync_copy(rows_v, out_hbm.at[pl.ds(wid * _TPW, _TPW), :])


@jax.jit
def kernel(x, W, b):
    wt = W.T
    b2 = b.reshape(1, _NUM_EXPERTS)
    pk0 = _router(x, wt, b2, 0)
    pk1 = _router(x, wt, b2, 1)
    o0 = _sc_expand(pk0)
    o1 = _sc_expand(pk1)
    return jnp.concatenate([o0, o1], axis=0)


# speedup vs baseline: 2.4515x; 2.4515x over previous
import jax
import jax.numpy as jnp
from jax.experimental import pallas as pl
from jax.experimental.pallas import tpu as pltpu

_INPUT_DIM = 768
_NUM_EXPERTS = 64
_BLOCK_T = 8192
_CHUNK = 256


def _dot(a, bm):
    return jax.lax.dot_general(
        a, bm,
        dimension_numbers=(((1,), (0,)), ((), ())),
        preferred_element_type=jnp.float32,
    )


def _gating_block(x0_ref, x1_ref, x2_ref, wt_ref, b_ref, o_ref):
    w = wt_ref[...]
    logits = (_dot(x0_ref[...], w[0:_CHUNK, :])
              + _dot(x1_ref[...], w[_CHUNK:2 * _CHUNK, :])
              + _dot(x2_ref[...], w[2 * _CHUNK:3 * _CHUNK, :])
              + b_ref[...])
    v1 = jnp.max(logits, axis=1, keepdims=True)
    m1 = logits == v1
    masked = jnp.where(m1, -jnp.inf, logits)
    v2 = jnp.max(masked, axis=1, keepdims=True)
    m2 = masked == v2
    t = jnp.exp(v2 - v1)
    w1 = 1.0 / (1.0 + t)
    w2 = t * w1
    o_ref[...] = jnp.where(m1, w1, 0.0) + jnp.where(m2, w2, 0.0)


@jax.jit
def kernel(x, W, b):
    tokens = x.shape[0]
    wt = W.T
    b2 = b.reshape(1, _NUM_EXPERTS)
    grid = (tokens // _BLOCK_T,)
    xspec = lambda j: pl.BlockSpec((_BLOCK_T, _CHUNK), lambda i, j=j: (i, j))
    return pl.pallas_call(
        _gating_block,
        grid=grid,
        in_specs=[
            xspec(0),
            xspec(1),
            xspec(2),
            pl.BlockSpec((_INPUT_DIM, _NUM_EXPERTS), lambda i: (0, 0)),
            pl.BlockSpec((1, _NUM_EXPERTS), lambda i: (0, 0)),
        ],
        out_specs=pl.BlockSpec((_BLOCK_T, _NUM_EXPERTS), lambda i: (i, 0)),
        out_shape=jax.ShapeDtypeStruct((tokens, _NUM_EXPERTS), jnp.float32),
        compiler_params=pltpu.CompilerParams(
            dimension_semantics=("parallel",),
            vmem_limit_bytes=64 * 1024 * 1024,
        ),
    )(x, x, x, wt, b2)


# speedup vs baseline: 2.5207x; 1.0282x over previous
import jax
import jax.numpy as jnp
from jax.experimental import pallas as pl
from jax.experimental.pallas import tpu as pltpu

_INPUT_DIM = 768
_NUM_EXPERTS = 64
_BLOCK_T = 4096
_CHUNK = 256


def _dot(a, bm):
    return jax.lax.dot_general(
        a, bm,
        dimension_numbers=(((1,), (0,)), ((), ())),
        preferred_element_type=jnp.float32,
    )


def _gating_block(x0_ref, x1_ref, x2_ref, wt_ref, b_ref, o_ref):
    w = wt_ref[...]
    logits = (_dot(x0_ref[...], w[0:_CHUNK, :])
              + _dot(x1_ref[...], w[_CHUNK:2 * _CHUNK, :])
              + _dot(x2_ref[...], w[2 * _CHUNK:3 * _CHUNK, :])
              + b_ref[...])
    v1 = jnp.max(logits, axis=1, keepdims=True)
    m1 = logits == v1
    masked = jnp.where(m1, -jnp.inf, logits)
    v2 = jnp.max(masked, axis=1, keepdims=True)
    m2 = masked == v2
    t = jnp.exp(v2 - v1)
    w1 = 1.0 / (1.0 + t)
    w2 = t * w1
    o_ref[...] = jnp.where(m1, w1, 0.0) + jnp.where(m2, w2, 0.0)


@jax.jit
def kernel(x, W, b):
    tokens = x.shape[0]
    wt = W.T
    b2 = b.reshape(1, _NUM_EXPERTS)
    grid = (tokens // _BLOCK_T,)
    xspec = lambda j: pl.BlockSpec((_BLOCK_T, _CHUNK), lambda i, j=j: (i, j))
    return pl.pallas_call(
        _gating_block,
        grid=grid,
        in_specs=[
            xspec(0),
            xspec(1),
            xspec(2),
            pl.BlockSpec((_INPUT_DIM, _NUM_EXPERTS), lambda i: (0, 0)),
            pl.BlockSpec((1, _NUM_EXPERTS), lambda i: (0, 0)),
        ],
        out_specs=pl.BlockSpec((_BLOCK_T, _NUM_EXPERTS), lambda i: (i, 0)),
        out_shape=jax.ShapeDtypeStruct((tokens, _NUM_EXPERTS), jnp.float32),
        compiler_params=pltpu.CompilerParams(
            dimension_semantics=("parallel",),
            vmem_limit_bytes=64 * 1024 * 1024,
        ),
    )(x, x, x, wt, b2)
